# Initial kernel scaffold; baseline (speedup 1.0000x reference)
import jax
import jax.numpy as jnp
from jax.experimental import pallas as pl


def kernel(input_spikes, weight, bias):
    raise NotImplementedError("write your pallas kernel here")


# speedup vs baseline: 3.5188x; 3.5188x over previous
import functools

import jax
import jax.numpy as jnp
from jax.experimental import pallas as pl
from jax.experimental.pallas import tpu as pltpu

_IN_C = 2
_OUT_C = 32
_K = 3
_STRIDE = 2
_STEP = 16
_LEAK = 32
_KSIZE = _STEP + _LEAK
_FODEP = _KSIZE
_DENSE = 0.3
_THETA = _DENSE * (_K * _K * _IN_C)
_WINNERS = 0.5
_B = 4
_XY = 48
_T = 48
_NX = (_XY - _K) // _STRIDE + 1
_N = _NX * _NX
_NPAD = 640
_TOUT = _T + 2 * _KSIZE - _KSIZE + 1
_CK = _IN_C * _K * _K
_KDIM = _CK * _KSIZE
_MDIM = _TOUT * _OUT_C
_NDIM = _B * _NPAD
_WINNER_FODEP = int(-(-_WINNERS * _N // 1))


_CHUNK = 16


def _fused_kernel(u_ref, wbig_ref, bias_ref, tau_ref, t_ref, out_ref):
    wbig = wbig_ref[:]
    bias = bias_ref[:]
    t_row = t_ref[:]
    u = u_ref[:]
    cix = jax.lax.broadcasted_iota(jnp.int32, (_OUT_C, _NDIM), 0)
    dep = jnp.zeros((1, _NDIM), dtype=jnp.int32)
    for c0 in range(0, _TOUT, _CHUNK):
        nt = min(_CHUNK, _TOUT - c0)
        tau_chunk = tau_ref[c0 * _OUT_C:(c0 + nt) * _OUT_C, :]
        s = tau_chunk - 1.0 - t_row
        wc = jnp.broadcast_to(wbig[None], (nt, _OUT_C, _KDIM)).reshape(
            nt * _OUT_C, _KDIM)
        f = jnp.minimum(s * (1.0 / _STEP), 1.5 * wc - s * (1.0 / _LEAK))
        f = jnp.where(s >= 0.0, jnp.maximum(f, 0.0), 0.0)
        pot = jax.lax.dot_general(
            f, u, (((1,), (0,)), ((), ())),
            preferred_element_type=jnp.float32)
        for j in range(nt):
            pt = pot[j * _OUT_C:(j + 1) * _OUT_C, :] + bias
            alive = (dep == 0)
            kok = []
            for b in range(_B):
                cnt = jnp.sum((dep[:, b * _NPAD:(b + 1) * _NPAD] != 0)
                              .astype(jnp.float32))
                ok = (cnt < float(_WINNER_FODEP)).astype(jnp.float32)
                kok.append(jnp.broadcast_to(ok.reshape(1, 1), (1, _NPAD)))
            mfac = alive.astype(jnp.float32) * jnp.concatenate(kok, axis=1)
            masked = pt * mfac
            m = jnp.max(masked, axis=0, keepdims=True)
            eq = masked == m
            winner = jnp.min(jnp.where(eq, cix, _OUT_C * 2), axis=0,
                             keepdims=True)
            spike = m > _THETA
            onehot = jnp.where((cix == winner) & spike, 1.0, 0.0)
            out_ref[c0 + j, :, :] = onehot
            dep = jnp.clip(dep + jnp.where(spike, _FODEP, 0) - 1,
                           0, _FODEP - 1)


@jax.jit
def kernel(input_spikes, weight, bias):
    cols = []
    for i in range(_IN_C):
        for kx in range(_K):
            for ky in range(_K):
                cols.append(jax.lax.slice(
                    input_spikes,
                    (0, i, kx, ky, 0),
                    (_B, i + 1, kx + 2 * (_NX - 1) + 1,
                     ky + 2 * (_NX - 1) + 1, _T),
                    (1, 1, _STRIDE, _STRIDE, 1)))
    a = jnp.concatenate(cols, axis=1)
    a = a.reshape(_B, _CK, _N, _T)
    a = jnp.pad(a, ((0, 0), (0, 0), (0, _NPAD - _N), (0, 0)))
    u = jnp.transpose(a, (1, 3, 0, 2)).reshape(_KDIM, _NDIM)

    wbig = jnp.repeat(weight.reshape(_OUT_C, _CK), _KSIZE, axis=1)
    bias2 = bias.reshape(_OUT_C, 1)
    tau_col = (jnp.arange(_MDIM, dtype=jnp.float32) // _OUT_C).reshape(
        _MDIM, 1)
    t_row = (jnp.arange(_KDIM, dtype=jnp.float32) % _KSIZE).reshape(1, _KDIM)

    out = pl.pallas_call(
        _fused_kernel,
        out_shape=jax.ShapeDtypeStruct((_TOUT, _OUT_C, _NDIM), jnp.float32),
    )(u, wbig, bias2, tau_col, t_row)

    spikes = out.reshape(_TOUT, _OUT_C, _B, _NPAD)[:, :, :, :_N]
    spikes = jnp.transpose(spikes, (2, 1, 3, 0))
    return spikes.reshape(_B, _OUT_C, _NX, _NX, _TOUT)


# speedup vs baseline: 4.0964x; 1.1642x over previous
import functools

import jax
import jax.numpy as jnp
from jax.experimental import pallas as pl
from jax.experimental.pallas import tpu as pltpu

_IN_C = 2
_OUT_C = 32
_K = 3
_STRIDE = 2
_STEP = 16
_LEAK = 32
_KSIZE = _STEP + _LEAK
_FODEP = _KSIZE
_DENSE = 0.3
_THETA = _DENSE * (_K * _K * _IN_C)
_WINNERS = 0.5
_B = 4
_XY = 48
_T = 48
_NX = (_XY - _K) // _STRIDE + 1
_N = _NX * _NX
_NPAD = 640
_TOUT = _T + 2 * _KSIZE - _KSIZE + 1
_CK = _IN_C * _K * _K
_KDIM = _CK * _KSIZE
_MDIM = _TOUT * _OUT_C
_NDIM = _B * _NPAD
_WINNER_FODEP = int(-(-_WINNERS * _N // 1))


_CHUNK = 16


def _fused_kernel(u_ref, wbig_ref, bias_ref, tau_ref, t_ref, out_ref,
                  code_ref):
    wbig = wbig_ref[:]
    bias = bias_ref[:]
    t_row = t_ref[:]
    u = u_ref[:]
    cix = jax.lax.broadcasted_iota(jnp.int32, (_OUT_C, _NDIM), 0)
    dep = jnp.zeros((1, _NDIM), dtype=jnp.int32)
    for c0 in range(0, _TOUT, _CHUNK):
        nt = min(_CHUNK, _TOUT - c0)
        tau_chunk = tau_ref[c0 * _OUT_C:(c0 + nt) * _OUT_C, :]
        s = tau_chunk - 1.0 - t_row
        wc = jnp.broadcast_to(wbig[None], (nt, _OUT_C, _KDIM)).reshape(
            nt * _OUT_C, _KDIM)
        f = jnp.minimum(s * (1.0 / _STEP), 1.5 * wc - s * (1.0 / _LEAK))
        f = jnp.where(s >= 0.0, jnp.maximum(f, 0.0), 0.0)
        pot = jax.lax.dot_general(
            f, u, (((1,), (0,)), ((), ())),
            preferred_element_type=jnp.float32)
        for j in range(nt):
            pt = pot[j * _OUT_C:(j + 1) * _OUT_C, :] + bias
            alive = (dep == 0)
            kok = []
            for b in range(_B):
                cnt = jnp.sum((dep[:, b * _NPAD:(b + 1) * _NPAD] != 0)
                              .astype(jnp.float32))
                ok = (cnt < float(_WINNER_FODEP)).astype(jnp.float32)
                kok.append(jnp.broadcast_to(ok.reshape(1, 1), (1, _NPAD)))
            mfac = alive.astype(jnp.float32) * jnp.concatenate(kok, axis=1)
            masked = pt * mfac
            m = jnp.max(masked, axis=0, keepdims=True)
            eq = masked == m
            winner = jnp.min(jnp.where(eq, cix, _OUT_C * 2), axis=0,
                             keepdims=True)
            spike = m > _THETA
            code_ref[pl.ds(c0 + j, 1), :] = jnp.where(spike, winner, -1)
            dep = jnp.clip(dep + jnp.where(spike, _FODEP, 0) - 1,
                           0, _FODEP - 1)
    code_t = jnp.transpose(code_ref[:, :])
    code_r = code_t.reshape(_B, _NPAD, _TOUT)[:, :_N, :]
    oix = jax.lax.broadcasted_iota(jnp.int32, (_B, _OUT_C, _N, _TOUT), 1)
    out_ref[:] = (code_r[:, None, :, :] == oix).astype(jnp.float32)


@jax.jit
def kernel(input_spikes, weight, bias):
    cols = []
    for i in range(_IN_C):
        for kx in range(_K):
            for ky in range(_K):
                cols.append(jax.lax.slice(
                    input_spikes,
                    (0, i, kx, ky, 0),
                    (_B, i + 1, kx + 2 * (_NX - 1) + 1,
                     ky + 2 * (_NX - 1) + 1, _T),
                    (1, 1, _STRIDE, _STRIDE, 1)))
    a = jnp.concatenate(cols, axis=1)
    a = a.reshape(_B, _CK, _N, _T)
    a = jnp.pad(a, ((0, 0), (0, 0), (0, _NPAD - _N), (0, 0)))
    u = jnp.transpose(a, (1, 3, 0, 2)).reshape(_KDIM, _NDIM)

    wbig = jnp.repeat(weight.reshape(_OUT_C, _CK), _KSIZE, axis=1)
    bias2 = bias.reshape(_OUT_C, 1)
    tau_col = (jnp.arange(_MDIM, dtype=jnp.float32) // _OUT_C).reshape(
        _MDIM, 1)
    t_row = (jnp.arange(_KDIM, dtype=jnp.float32) % _KSIZE).reshape(1, _KDIM)

    out = pl.pallas_call(
        _fused_kernel,
        out_shape=jax.ShapeDtypeStruct((_B, _OUT_C, _N, _TOUT), jnp.float32),
        scratch_shapes=[pltpu.VMEM((_TOUT, _NDIM), jnp.int32)],
    )(u, wbig, bias2, tau_col, t_row)

    return out.reshape(_B, _OUT_C, _NX, _NX, _TOUT)


# speedup vs baseline: 34.7673x; 8.4873x over previous
import jax
import jax.numpy as jnp
from jax.experimental import pallas as pl
from jax.experimental.pallas import tpu as pltpu

_IN_C = 2
_OUT_C = 32
_K = 3
_STRIDE = 2
_STEP = 16
_LEAK = 32
_KSIZE = _STEP + _LEAK
_FODEP = _KSIZE
_DENSE = 0.3
_THETA = _DENSE * (_K * _K * _IN_C)
_WINNERS = 0.5
_B = 4
_XY = 48
_T = 48
_NX = (_XY - _K) // _STRIDE + 1
_N = _NX * _NX
_HPAD = 24
_WPAD = 24
_NPAD = _HPAD * _WPAD
_TOUT = _T + 2 * _KSIZE - _KSIZE + 1
_CK = _IN_C * _K * _K
_KDIM = _CK * _KSIZE
_NDIM = _B * _NPAD
_WINNER_FODEP = int(-(-_WINNERS * _N // 1))
_CHUNK = 8


def _fused_kernel(x_ref, wbig_ref, bias_ref, t_ref, out_ref,
                  ut_ref, code_ref):
    ut_ref[...] = jnp.zeros((_B, _HPAD, _WPAD, _KDIM), jnp.float32)
    for i in range(_IN_C):
        for kx in range(_K):
            for ky in range(_K):
                c = i * _K * _K + kx * _K + ky
                h0, w0 = kx // 2, ky // 2
                t0 = (ky % 2) * _KSIZE
                sl = x_ref[:, i, kx % 2, h0:h0 + _NX, w0:w0 + _NX,
                           t0:t0 + _KSIZE]
                ut_ref[:, :_NX, :_NX, c * _KSIZE:(c + 1) * _KSIZE] = sl

    wbig = wbig_ref[:]
    bias = bias_ref[:]
    t_row = t_ref[:]
    cix = jax.lax.broadcasted_iota(jnp.int32, (_OUT_C, _NDIM), 0)
    dep = jnp.zeros((1, _NDIM), dtype=jnp.int32)
    for c0 in range(0, _TOUT, _CHUNK):
        nt = min(_CHUNK, _TOUT - c0)
        tau_local = jax.lax.broadcasted_iota(
            jnp.int32, (nt, _OUT_C, 1), 0).reshape(
                nt * _OUT_C, 1).astype(jnp.float32)
        s = (float(c0) - 1.0 + tau_local) - t_row
        wc = jnp.broadcast_to(wbig[None], (nt, _OUT_C, _KDIM)).reshape(
            nt * _OUT_C, _KDIM)
        f = jnp.minimum(s * (1.0 / _STEP), 1.5 * wc - s * (1.0 / _LEAK))
        f = jnp.where(s >= 0.0, jnp.maximum(f, 0.0), 0.0)
        ut = ut_ref[...].reshape(_NDIM, _KDIM)
        pot = jax.lax.dot_general(
            f, ut, (((1,), (1,)), ((), ())),
            preferred_element_type=jnp.float32)
        for j in range(nt):
            pt = pot[j * _OUT_C:(j + 1) * _OUT_C, :] + bias
            alive = (dep == 0)
            kok = []
            for b in range(_B):
                cnt = jnp.sum((dep[:, b * _NPAD:(b + 1) * _NPAD] != 0)
                              .astype(jnp.float32))
                ok = (cnt < float(_WINNER_FODEP)).astype(jnp.float32)
                kok.append(jnp.broadcast_to(ok.reshape(1, 1), (1, _NPAD)))
            mfac = alive.astype(jnp.float32) * jnp.concatenate(kok, axis=1)
            masked = pt * mfac
            m = jnp.max(masked, axis=0, keepdims=True)
            eq = masked == m
            winner = jnp.min(jnp.where(eq, cix, _OUT_C * 2), axis=0,
                             keepdims=True)
            spike = m > _THETA
            code_ref[pl.ds(c0 + j, 1), :] = jnp.where(spike, winner, -1)
            dep = jnp.clip(dep + jnp.where(spike, _FODEP, 0) - 1,
                           0, _FODEP - 1)
    code_t = jnp.transpose(code_ref[:, :])
    code_r = code_t.reshape(_B, _HPAD, _WPAD, _TOUT)[:, :_NX, :_NX, :]
    oix = jax.lax.broadcasted_iota(
        jnp.int32, (_B, _OUT_C, _NX, _NX, _TOUT), 1)
    out_ref[:] = (code_r[:, None, :, :, :] == oix).astype(jnp.float32)


@jax.jit
def kernel(input_spikes, weight, bias):
    xp = jnp.transpose(
        input_spikes.reshape(_B, _IN_C, _XY // 2, 2, _XY // 2, 2, _T),
        (0, 1, 3, 2, 4, 5, 6)).reshape(
            _B, _IN_C, 2, _XY // 2, _XY // 2, 2 * _T)
    wbig = jnp.repeat(weight.reshape(_OUT_C, _CK), _KSIZE, axis=1)
    bias2 = bias.reshape(_OUT_C, 1)
    t_row = (jnp.arange(_KDIM, dtype=jnp.float32) % _KSIZE).reshape(1, _KDIM)

    out = pl.pallas_call(
        _fused_kernel,
        out_shape=jax.ShapeDtypeStruct(
            (_B, _OUT_C, _NX, _NX, _TOUT), jnp.float32),
        scratch_shapes=[pltpu.VMEM((_B, _HPAD, _WPAD, _KDIM), jnp.float32),
                        pltpu.VMEM((_TOUT, _NDIM), jnp.int32)],
    )(xp, wbig, bias2, t_row)

    return out
